# Initial kernel scaffold; baseline (speedup 1.0000x reference)
import jax
import jax.numpy as jnp
from jax.experimental import pallas as pl


def kernel(x, edge_index_r0, edge_index_r1, W1_r0, b1_r0, W1_r1, b1_r1, W2_r0, b2_r0, W2_r1, b2_r1, W3_r0, b3_r0, W3_r1, b3_r1, Wd, bd):
    raise NotImplementedError("write your pallas kernel here")


# speedup vs baseline: 30.9655x; 30.9655x over previous
import functools

import jax
import jax.numpy as jnp
from jax import lax
from jax.experimental import pallas as pl
from jax.experimental.pallas import tpu as pltpu
from jax.experimental.pallas import tpu_sc as plsc

_N = 10000
_E = 320000
_NT = 16
_NA = 10240
_SL = _NA // _NT
_ROWS = 157
_EC = _ROWS * 128
_EPT = _E // _NT
_HI = lax.Precision.HIGHEST


def _rsqrt16(x):
    xb = lax.bitcast_convert_type(x, jnp.int32)
    y = lax.bitcast_convert_type(jnp.int32(0x5F3759DF) - (xb >> 1), jnp.float32)
    for _ in range(3):
        y = y * (1.5 - 0.5 * x * y * y)
    return y


def _prep(ei):
    src = ei[0].reshape(_NT, _EPT)
    dst = ei[1].reshape(_NT, _EPT)
    pad_s = jnp.full((_NT, _EC - _EPT), _N, jnp.int32)
    pad_d = jnp.zeros((_NT, _EC - _EPT), jnp.int32)
    src = jnp.concatenate([src, pad_s], axis=1)
    dst = jnp.concatenate([dst, pad_d], axis=1)
    return src, dst


def _sc_backprop(src0, dst0, src1, dst1):
    mesh = plsc.VectorSubcoreMesh(
        core_axis_name="c", subcore_axis_name="s", num_cores=1)

    @functools.partial(
        pl.kernel,
        out_type=jax.ShapeDtypeStruct((14, _NA), jnp.float32),
        mesh=mesh,
        scratch_types=[
            pltpu.VMEM((_EC,), jnp.int32),
            pltpu.VMEM((_EC,), jnp.int32),
            pltpu.VMEM((_EC,), jnp.int32),
            pltpu.VMEM((_EC,), jnp.int32),
            pltpu.VMEM((_EC,), jnp.float32),
            pltpu.VMEM((_SL,), jnp.float32),
            pltpu.VMEM((_SL,), jnp.float32),
            pltpu.VMEM((_SL,), jnp.float32),
        ] + [pltpu.VMEM_SHARED((_NA,), jnp.float32) for _ in range(18)],
    )
    def k(src0_h, dst0_h, src1_h, dst1_h, out_h,
          sidx0, didx0, sidx1, didx1, vals, wa, wb, zer,
          ns0, nd0, ns1, nd1, a10, a11, a20, a21, a22, a23,
          u0, u1, u2, u3, ac0, ac1, ac2, ac3):
        sid = lax.axis_index("s")
        sl = pl.ds(sid * _SL, _SL)

        pltpu.sync_copy(src0_h.at[sid], sidx0)
        pltpu.sync_copy(dst0_h.at[sid], didx0)
        pltpu.sync_copy(src1_h.at[sid], sidx1)
        pltpu.sync_copy(dst1_h.at[sid], didx1)

        def fill1d(ref, v):
            def bb(i, _):
                ref[pl.ds(pl.multiple_of(i * 16, 16), 16)] = jnp.full(
                    (16,), v, jnp.float32)
                return 0
            lax.fori_loop(0, _SL // 16, bb, 0)

        def fill_edges(ref, v):
            def bb(i, _):
                ref[pl.ds(pl.multiple_of(i * 16, 16), 16)] = jnp.full(
                    (16,), v, jnp.float32)
                return 0
            lax.fori_loop(0, _EC // 16, bb, 0)

        def zero(sp):
            pltpu.sync_copy(zer, sp.at[sl])

        def rsqrt_arr(deg_sp, out_sp):
            pltpu.sync_copy(deg_sp.at[sl], wa)
            def bb(i, _):
                s = pl.ds(pl.multiple_of(i * 16, 16), 16)
                wb[s] = _rsqrt16(jnp.maximum(wa[s], 1.0))
                return 0
            lax.fori_loop(0, _SL // 16, bb, 0)
            pltpu.sync_copy(wb, out_sp.at[sl])

        def mul_store(a_sp, b_sp, dst_sp=None, row=None):
            pltpu.sync_copy(a_sp.at[sl], wa)
            pltpu.sync_copy(b_sp.at[sl], wb)
            def bb(i, _):
                s = pl.ds(pl.multiple_of(i * 16, 16), 16)
                wa[s] = wa[s] * wb[s]
                return 0
            lax.fori_loop(0, _SL // 16, bb, 0)
            if dst_sp is not None:
                pltpu.sync_copy(wa, dst_sp.at[sl])
            if row is not None:
                pltpu.sync_copy(wa, out_h.at[row, sl])

        def prop(didx, sidx, u_sp, acc_sp):
            pltpu.sync_copy(u_sp.at[didx], vals)
            pltpu.sync_copy(vals, acc_sp.at[sidx], add=True)

        fill1d(zer, 0.0)
        fill_edges(vals, 1.0)
        for a in (ac0, ac1, ac2, ac3):
            zero(a)
        plsc.subcore_barrier()

        pltpu.sync_copy(vals, ac0.at[sidx0], add=True)
        pltpu.sync_copy(vals, ac1.at[didx0], add=True)
        pltpu.sync_copy(vals, ac2.at[sidx1], add=True)
        pltpu.sync_copy(vals, ac3.at[didx1], add=True)
        plsc.subcore_barrier()
        rsqrt_arr(ac0, ns0)
        rsqrt_arr(ac1, nd0)
        rsqrt_arr(ac2, ns1)
        rsqrt_arr(ac3, nd1)
        zero(ac0)
        zero(ac1)
        plsc.subcore_barrier()

        prop(didx0, sidx0, nd0, ac0)
        prop(didx1, sidx1, nd1, ac1)
        plsc.subcore_barrier()
        mul_store(ac0, ns0, a10, 0)
        mul_store(ac1, ns1, a11, 1)

        mul_store(a10, nd0, u0)
        mul_store(a11, nd0, u1)
        mul_store(a10, nd1, u2)
        mul_store(a11, nd1, u3)
        for a in (ac0, ac1, ac2, ac3):
            zero(a)
        plsc.subcore_barrier()
        prop(didx0, sidx0, u0, ac0)
        prop(didx0, sidx0, u1, ac1)
        prop(didx1, sidx1, u2, ac2)
        prop(didx1, sidx1, u3, ac3)
        plsc.subcore_barrier()
        mul_store(ac0, ns0, a20, 2)
        mul_store(ac1, ns0, a21, 3)
        mul_store(ac2, ns1, a22, 4)
        mul_store(ac3, ns1, a23, 5)

        mul_store(a20, nd0, u0)
        mul_store(a21, nd0, u1)
        mul_store(a22, nd0, u2)
        mul_store(a23, nd0, u3)
        for a in (ac0, ac1, ac2, ac3):
            zero(a)
        plsc.subcore_barrier()
        prop(didx0, sidx0, u0, ac0)
        prop(didx0, sidx0, u1, ac1)
        prop(didx0, sidx0, u2, ac2)
        prop(didx0, sidx0, u3, ac3)
        plsc.subcore_barrier()
        mul_store(ac0, ns0, None, 6)
        mul_store(ac1, ns0, None, 7)
        mul_store(ac2, ns0, None, 8)
        mul_store(ac3, ns0, None, 9)

        mul_store(a20, nd1, u0)
        mul_store(a21, nd1, u1)
        mul_store(a22, nd1, u2)
        mul_store(a23, nd1, u3)
        for a in (ac0, ac1, ac2, ac3):
            zero(a)
        plsc.subcore_barrier()
        prop(didx1, sidx1, u0, ac0)
        prop(didx1, sidx1, u1, ac1)
        prop(didx1, sidx1, u2, ac2)
        prop(didx1, sidx1, u3, ac3)
        plsc.subcore_barrier()
        mul_store(ac0, ns1, None, 10)
        mul_store(ac1, ns1, None, 11)
        mul_store(ac2, ns1, None, 12)
        mul_store(ac3, ns1, None, 13)

    return k(src0, dst0, src1, dst1)


def _combine_tc(x, avec, w1a, w1b, w2a, w2b, w3a, w3b, wd, b1s, b2s, b3s, bds):
    def body(x_ref, a_ref, w1a_ref, w1b_ref, w2a_ref, w2b_ref,
             w3a_ref, w3b_ref, wd_ref, b1_ref, b2_ref, b3_ref, bd_ref, o_ref):
        xv = x_ref[...]
        av = a_ref[...]
        z = lax.dot_general(av[6:14, :_N], xv, (((1,), (0,)), ((), ())),
                            preferred_element_type=jnp.float32,
                            precision=_HI)
        s1 = jnp.sum(av[0:2, :_N], axis=1)
        s2 = jnp.sum(av[2:6, :_N], axis=1)
        wdv = wd_ref[...]
        g = [jnp.dot(w3a_ref[...], wdv, precision=_HI),
             jnp.dot(w3b_ref[...], wdv, precision=_HI)]
        w2 = [w2a_ref[...], w2b_ref[...]]
        q = [[jnp.dot(w2[s], g[r], precision=_HI) for r in (0, 1)]
             for s in (0, 1)]
        core = jnp.float32(0.0)
        kk = 0
        for w1_ref in (w1a_ref, w1b_ref):
            for s in (0, 1):
                for r in (0, 1):
                    mv = jnp.dot(w1_ref[...], q[s][r], precision=_HI)
                    core = core + jnp.dot(z[kk:kk + 1, :], mv,
                                          precision=_HI)[0, 0]
                    kk += 1
        bias = jnp.float32(_N) * jnp.dot(b3_ref[...], wdv, precision=_HI)[0, 0]
        for s in (0, 1):
            for r in (0, 1):
                bias = bias + s2[2 * s + r] * jnp.dot(
                    b1_ref[...], q[s][r], precision=_HI)[0, 0]
        for r in (0, 1):
            bias = bias + s1[r] * jnp.dot(
                b2_ref[...], g[r], precision=_HI)[0, 0]
        out = (core + bias) / jnp.float32(_N) + bd_ref[0, 0]
        o_ref[...] = out.reshape(1, 1)

    return pl.pallas_call(
        body,
        out_shape=jax.ShapeDtypeStruct((1, 1), jnp.float32),
    )(x, avec, w1a, w1b, w2a, w2b, w3a, w3b, wd, b1s, b2s, b3s, bds)


def kernel(x, edge_index_r0, edge_index_r1,
           W1_r0, b1_r0, W1_r1, b1_r1,
           W2_r0, b2_r0, W2_r1, b2_r1,
           W3_r0, b3_r0, W3_r1, b3_r1,
           Wd, bd):
    s0, d0 = _prep(edge_index_r0)
    s1, d1 = _prep(edge_index_r1)
    avec = _sc_backprop(s0, d0, s1, d1)
    b1s = (b1_r0 + b1_r1).reshape(1, 256)
    b2s = (b2_r0 + b2_r1).reshape(1, 128)
    b3s = (b3_r0 + b3_r1).reshape(1, 64)
    return _combine_tc(x, avec, W1_r0, W1_r1, W2_r0, W2_r1, W3_r0, W3_r1,
                       Wd, b1s, b2s, b3s, bd.reshape(1, 1))


# speedup vs baseline: 35.9722x; 1.1617x over previous
import functools

import jax
import jax.numpy as jnp
from jax import lax
from jax.experimental import pallas as pl
from jax.experimental.pallas import tpu as pltpu
from jax.experimental.pallas import tpu_sc as plsc

_N = 10000
_E = 320000
_NT = 16
_NA = 10240
_SL = _NA // _NT
_EC = _E // _NT
_HI = lax.Precision.HIGHEST


def _rsqrt16(x):
    xb = lax.bitcast_convert_type(x, jnp.int32)
    y = lax.bitcast_convert_type(jnp.int32(0x5F3759DF) - (xb >> 1), jnp.float32)
    for _ in range(3):
        y = y * (1.5 - 0.5 * x * y * y)
    return y


def _sc_backprop(src0, dst0, src1, dst1):
    mesh = plsc.VectorSubcoreMesh(
        core_axis_name="c", subcore_axis_name="s", num_cores=1)

    @functools.partial(
        pl.kernel,
        out_type=jax.ShapeDtypeStruct((14, _NA), jnp.float32),
        mesh=mesh,
        scratch_types=[
            pltpu.VMEM((_EC,), jnp.int32),
            pltpu.VMEM((_EC,), jnp.int32),
            pltpu.VMEM((_EC,), jnp.int32),
            pltpu.VMEM((_EC,), jnp.int32),
            pltpu.VMEM((_EC,), jnp.float32),
            pltpu.VMEM((_EC,), jnp.float32),
            pltpu.VMEM((_SL,), jnp.float32),
            pltpu.VMEM((_SL,), jnp.float32),
            pltpu.VMEM((_SL,), jnp.float32),
            pltpu.VMEM((_SL,), jnp.float32),
            pltpu.VMEM((_SL,), jnp.float32),
            pltpu.VMEM((_SL,), jnp.float32),
            pltpu.VMEM((_SL,), jnp.float32),
            pltpu.VMEM((_SL,), jnp.float32),
            pltpu.VMEM((_SL,), jnp.float32),
        ] + [pltpu.VMEM_SHARED((_NA,), jnp.float32) for _ in range(10)]
        + [pltpu.SemaphoreType.DMA for _ in range(4)],
    )
    def k(src0_h, dst0_h, src1_h, dst1_h, out_h,
          sidx0, didx0, sidx1, didx1, valsA, valsB, wa, wb, zer,
          ns0, ns1, a20, a21, a22, a23,
          nd0, nd1, u0, u1, u2, u3, ac0, ac1, ac2, ac3,
          sga, sgb, ssa, ssb):
        sid = lax.axis_index("s")
        sl = pl.ds(sid * _SL, _SL)

        st0 = pltpu.async_copy(src0_h.at[sid], sidx0, sga)
        st1 = pltpu.async_copy(dst0_h.at[sid], didx0, sgb)
        st2 = pltpu.async_copy(src1_h.at[sid], sidx1, ssa)
        st3 = pltpu.async_copy(dst1_h.at[sid], didx1, ssb)

        def fill1d(ref, n, v):
            def bb(i, _):
                ref[pl.ds(pl.multiple_of(i * 16, 16), 16)] = jnp.full(
                    (16,), v, jnp.float32)
                return 0
            lax.fori_loop(0, n // 16, bb, 0)

        def zero(sp):
            pltpu.sync_copy(zer, sp.at[sl])

        def ew_mul(dstv, av, bv):
            def bb(i, _):
                s = pl.ds(pl.multiple_of(i * 16, 16), 16)
                dstv[s] = av[s] * bv[s]
                return 0
            lax.fori_loop(0, _SL // 16, bb, 0)

        def rsqrt_v(deg_sp, outv):
            pltpu.sync_copy(deg_sp.at[sl], wa)
            def bb(i, _):
                s = pl.ds(pl.multiple_of(i * 16, 16), 16)
                outv[s] = _rsqrt16(jnp.maximum(wa[s], 1.0))
                return 0
            lax.fori_loop(0, _SL // 16, bb, 0)

        def rsqrt_sp(deg_sp, out_sp):
            rsqrt_v(deg_sp, wb)
            pltpu.sync_copy(wb, out_sp.at[sl])

        def scale_acc_out(acc_sp, nsv, dst_sp, row):
            pltpu.sync_copy(acc_sp.at[sl], wa)
            ew_mul(wa, wa, nsv)
            if dst_sp is not None:
                pltpu.sync_copy(wa, dst_sp.at[sl])
            pltpu.sync_copy(wa, out_h.at[row, sl])

        def scale_acc_local(acc_sp, nsv, a2v, row):
            pltpu.sync_copy(acc_sp.at[sl], wa)
            ew_mul(a2v, wa, nsv)
            pltpu.sync_copy(a2v, out_h.at[row, sl])

        def mul_sp(a_sp, b_sp, dst_sp):
            pltpu.sync_copy(a_sp.at[sl], wa)
            pltpu.sync_copy(b_sp.at[sl], wb)
            ew_mul(wa, wa, wb)
            pltpu.sync_copy(wa, dst_sp.at[sl])

        def mul_local_sp(a2v, nd_sp, u_sp):
            pltpu.sync_copy(nd_sp.at[sl], wb)
            ew_mul(wa, a2v, wb)
            pltpu.sync_copy(wa, u_sp.at[sl])

        def props(chans):
            for didx, sidx, u_sp, acc_sp in chans:
                pltpu.sync_copy(u_sp.at[didx], valsA)
                pltpu.sync_copy(valsA, acc_sp.at[sidx], add=True)

        fill1d(zer, _SL, 0.0)
        fill1d(valsA, _EC, 1.0)
        for a in (ac0, ac1, ac2, ac3):
            zero(a)
        st0.wait()
        st1.wait()
        st2.wait()
        st3.wait()
        plsc.subcore_barrier()

        h0 = pltpu.async_copy(valsA, ac0.at[sidx0], sga, add=True)
        h1 = pltpu.async_copy(valsA, ac1.at[didx0], sgb, add=True)
        h0.wait()
        h1.wait()
        h2 = pltpu.async_copy(valsA, ac2.at[sidx1], ssa, add=True)
        h3 = pltpu.async_copy(valsA, ac3.at[didx1], ssb, add=True)
        h2.wait()
        h3.wait()
        plsc.subcore_barrier()
        rsqrt_v(ac0, ns0)
        rsqrt_sp(ac1, nd0)
        rsqrt_v(ac2, ns1)
        rsqrt_sp(ac3, nd1)
        zero(ac0)
        zero(ac1)
        plsc.subcore_barrier()

        props([(didx0, sidx0, nd0, ac0), (didx1, sidx1, nd1, ac1)])
        plsc.subcore_barrier()
        scale_acc_out(ac0, ns0, u0, 0)
        scale_acc_out(ac1, ns1, u1, 1)

        mul_sp(u0, nd1, u2)
        mul_sp(u1, nd1, u3)
        mul_sp(u0, nd0, u0)
        mul_sp(u1, nd0, u1)
        for a in (ac0, ac1, ac2, ac3):
            zero(a)
        plsc.subcore_barrier()
        props([(didx0, sidx0, u0, ac0), (didx0, sidx0, u1, ac1),
               (didx1, sidx1, u2, ac2), (didx1, sidx1, u3, ac3)])
        plsc.subcore_barrier()
        scale_acc_local(ac0, ns0, a20, 2)
        scale_acc_local(ac1, ns0, a21, 3)
        scale_acc_local(ac2, ns1, a22, 4)
        scale_acc_local(ac3, ns1, a23, 5)

        mul_local_sp(a20, nd0, u0)
        mul_local_sp(a21, nd0, u1)
        mul_local_sp(a22, nd0, u2)
        mul_local_sp(a23, nd0, u3)
        for a in (ac0, ac1, ac2, ac3):
            zero(a)
        plsc.subcore_barrier()
        props([(didx0, sidx0, u0, ac0), (didx0, sidx0, u1, ac1),
               (didx0, sidx0, u2, ac2), (didx0, sidx0, u3, ac3)])
        plsc.subcore_barrier()
        scale_acc_out(ac0, ns0, None, 6)
        scale_acc_out(ac1, ns0, None, 7)
        scale_acc_out(ac2, ns0, None, 8)
        scale_acc_out(ac3, ns0, None, 9)

        mul_local_sp(a20, nd1, u0)
        mul_local_sp(a21, nd1, u1)
        mul_local_sp(a22, nd1, u2)
        mul_local_sp(a23, nd1, u3)
        for a in (ac0, ac1, ac2, ac3):
            zero(a)
        plsc.subcore_barrier()
        props([(didx1, sidx1, u0, ac0), (didx1, sidx1, u1, ac1),
               (didx1, sidx1, u2, ac2), (didx1, sidx1, u3, ac3)])
        plsc.subcore_barrier()
        scale_acc_out(ac0, ns1, None, 10)
        scale_acc_out(ac1, ns1, None, 11)
        scale_acc_out(ac2, ns1, None, 12)
        scale_acc_out(ac3, ns1, None, 13)

    return k(src0, dst0, src1, dst1)


def _combine_tc(x, avec, w1a, w1b, w2a, w2b, w3a, w3b, wd, b1s, b2s, b3s, bds):
    def body(x_ref, a_ref, w1a_ref, w1b_ref, w2a_ref, w2b_ref,
             w3a_ref, w3b_ref, wd_ref, b1_ref, b2_ref, b3_ref, bd_ref, o_ref):
        xv = x_ref[...]
        av = a_ref[...]
        z = lax.dot_general(av[6:14, :_N], xv, (((1,), (0,)), ((), ())),
                            preferred_element_type=jnp.float32,
                            precision=_HI)
        s1 = jnp.sum(av[0:2, :_N], axis=1)
        s2 = jnp.sum(av[2:6, :_N], axis=1)
        wdv = wd_ref[...]
        g = [jnp.dot(w3a_ref[...], wdv, precision=_HI),
             jnp.dot(w3b_ref[...], wdv, precision=_HI)]
        w2 = [w2a_ref[...], w2b_ref[...]]
        q = [[jnp.dot(w2[s], g[r], precision=_HI) for r in (0, 1)]
             for s in (0, 1)]
        core = jnp.float32(0.0)
        kk = 0
        for w1_ref in (w1a_ref, w1b_ref):
            for s in (0, 1):
                for r in (0, 1):
                    mv = jnp.dot(w1_ref[...], q[s][r], precision=_HI)
                    core = core + jnp.dot(z[kk:kk + 1, :], mv,
                                          precision=_HI)[0, 0]
                    kk += 1
        bias = jnp.float32(_N) * jnp.dot(b3_ref[...], wdv, precision=_HI)[0, 0]
        for s in (0, 1):
            for r in (0, 1):
                bias = bias + s2[2 * s + r] * jnp.dot(
                    b1_ref[...], q[s][r], precision=_HI)[0, 0]
        for r in (0, 1):
            bias = bias + s1[r] * jnp.dot(
                b2_ref[...], g[r], precision=_HI)[0, 0]
        out = (core + bias) / jnp.float32(_N) + bd_ref[0, 0]
        o_ref[...] = out.reshape(1, 1)

    return pl.pallas_call(
        body,
        out_shape=jax.ShapeDtypeStruct((1, 1), jnp.float32),
    )(x, avec, w1a, w1b, w2a, w2b, w3a, w3b, wd, b1s, b2s, b3s, bds)


def kernel(x, edge_index_r0, edge_index_r1,
           W1_r0, b1_r0, W1_r1, b1_r1,
           W2_r0, b2_r0, W2_r1, b2_r1,
           W3_r0, b3_r0, W3_r1, b3_r1,
           Wd, bd):
    s0 = edge_index_r0[0].reshape(_NT, _EC)
    d0 = edge_index_r0[1].reshape(_NT, _EC)
    s1 = edge_index_r1[0].reshape(_NT, _EC)
    d1 = edge_index_r1[1].reshape(_NT, _EC)
    avec = _sc_backprop(s0, d0, s1, d1)
    b1s = (b1_r0 + b1_r1).reshape(1, 256)
    b2s = (b2_r0 + b2_r1).reshape(1, 128)
    b3s = (b3_r0 + b3_r1).reshape(1, 64)
    return _combine_tc(x, avec, W1_r0, W1_r1, W2_r0, W2_r1, W3_r0, W3_r1,
                       Wd, b1s, b2s, b3s, bd.reshape(1, 1))
